# baseline (speedup 1.0000x reference)
import jax
import jax.numpy as jnp
from jax.experimental import pallas as pl
from jax.experimental.pallas import tpu as pltpu

_N, _D, _K = 8192, 64, 512
_CAP = _N // _K
_ITERS = 3
_CHUNK = 1024
_B = 256


def _kmeans_kernel(data_ref, cb0_ref, cb_ref, labels_ref,
                   dist_ref, sums_ref, lab_ref, tri_ref):
    lane = jax.lax.broadcasted_iota(jnp.int32, (1, _K), 1)
    riota = jax.lax.broadcasted_iota(jnp.int32, (_B, 1), 0)
    riota_b = jax.lax.broadcasted_iota(jnp.int32, (_B, _K), 0)
    tri_ref[:] = (jax.lax.broadcasted_iota(jnp.int32, (_B, _B), 0) >=
                  jax.lax.broadcasted_iota(jnp.int32, (_B, _B), 1)
                  ).astype(jnp.bfloat16)
    dist_ref[pl.ds(_N, _B), :] = jnp.zeros((_B, _K), jnp.float32)
    cb_ref[:] = cb0_ref[:]
    for _ in range(_ITERS):
        cb = cb_ref[:]
        b2 = jnp.sum(cb * cb, axis=1)[None, :]
        for c in range(_N // _CHUNK):
            rows = data_ref[pl.ds(c * _CHUNK, _CHUNK), :]
            a2 = jnp.sum(rows * rows, axis=1, keepdims=True)
            prod = jax.lax.dot_general(
                rows, cb, (((1,), (1,)), ((), ())),
                preferred_element_type=jnp.float32)
            d2 = a2 + b2 - 2.0 * prod
            dist_ref[pl.ds(c * _CHUNK, _CHUNK), :] = jnp.sqrt(
                jnp.maximum(d2, 0.0))

        def round_body(state):
            p, counts = state
            o = jnp.bitwise_and(p, 7)
            p0 = pl.multiple_of(p - o, 8)
            win = dist_ref[pl.ds(p0, _B), :]
            open_f = jnp.where(counts < _CAP, 0.0, jnp.float32(jnp.inf))
            masked = win + open_f
            idx = jnp.argmin(masked, axis=1).astype(jnp.int32)[:, None]
            valid = jnp.logical_and(riota >= o, (p0 + riota) < _N)
            oh_bool = jnp.logical_and(lane == idx, valid)
            onehot = jnp.where(oh_bool, 1.0, 0.0).astype(jnp.bfloat16)
            prefix = jax.lax.dot_general(
                tri_ref[:], onehot, (((1,), (0,)), ((), ())),
                preferred_element_type=jnp.float32)
            rem = (_CAP - counts).astype(jnp.float32)
            ovf_pos = jnp.where(prefix > rem, riota_b, _B)
            fb = jnp.min(jnp.where(oh_bool, ovf_pos, _B))
            old = lab_ref[pl.ds(p0, _B), :]
            lab_ref[pl.ds(p0, _B), :] = jnp.where(riota >= o, idx, old)
            commit = (riota < fb).astype(jnp.float32)
            dcount = jnp.sum(onehot.astype(jnp.float32) * commit,
                             axis=0, keepdims=True)
            return jnp.minimum(p0 + fb, _N), counts + dcount.astype(jnp.int32)

        def round_cond(state):
            return state[0] < _N

        jax.lax.while_loop(
            round_cond, round_body,
            (jnp.int32(0), jnp.zeros((1, _K), jnp.int32)))

        sums_ref[:] = jnp.zeros((_K, _D), jnp.float32)
        for c in range(_N // _CHUNK):
            lbl = lab_ref[pl.ds(c * _CHUNK, _CHUNK), :]
            onehot = (lbl == jax.lax.broadcasted_iota(
                jnp.int32, (_CHUNK, _K), 1)).astype(jnp.bfloat16)
            rows = data_ref[pl.ds(c * _CHUNK, _CHUNK), :]
            hi = rows.astype(jnp.bfloat16)
            r1 = rows - hi.astype(jnp.float32)
            mid = r1.astype(jnp.bfloat16)
            lo = (r1 - mid.astype(jnp.float32)).astype(jnp.bfloat16)
            acc = sums_ref[:]
            for part in (hi, mid, lo):
                acc = acc + jax.lax.dot_general(
                    onehot, part, (((0,), (0,)), ((), ())),
                    preferred_element_type=jnp.float32)
            sums_ref[:] = acc
        cb_ref[:] = sums_ref[:] / jnp.float32(_CAP)
    labels_ref[:] = lab_ref[pl.ds(0, _N), :]


def kernel(data):
    n = data.shape[0]
    perm = jax.random.permutation(jax.random.key(1), n)[:_K]
    cb0 = data[perm]
    cb, labels = pl.pallas_call(
        _kmeans_kernel,
        out_shape=[
            jax.ShapeDtypeStruct((_K, _D), jnp.float32),
            jax.ShapeDtypeStruct((_N, 1), jnp.int32),
        ],
        scratch_shapes=[
            pltpu.VMEM((_N + _B, _K), jnp.float32),
            pltpu.VMEM((_K, _D), jnp.float32),
            pltpu.VMEM((_N + _B, 1), jnp.int32),
            pltpu.VMEM((_B, _B), jnp.bfloat16),
        ],
    )(data, cb0)
    return cb, labels.reshape(n)


# speedup vs baseline: 1.5756x; 1.5756x over previous
import jax
import jax.numpy as jnp
from jax.experimental import pallas as pl
from jax.experimental.pallas import tpu as pltpu

_N, _D, _K = 8192, 64, 512
_CAP = _N // _K
_ITERS = 3
_CHUNK = 1024
_BW = 512


def _kmeans_kernel(data_ref, cb0_ref, cb_ref, labels_ref,
                   dist_ref, sums_ref, oh_ref, pre_ref, tri_ref):
    lane = jax.lax.broadcasted_iota(jnp.int32, (1, _K), 1)
    tri_ref[:] = (jax.lax.broadcasted_iota(jnp.int32, (_BW, _BW), 0) >=
                  jax.lax.broadcasted_iota(jnp.int32, (_BW, _BW), 1)
                  ).astype(jnp.bfloat16)
    cb_ref[:] = cb0_ref[:]
    capf = jnp.float32(_CAP)
    for _ in range(_ITERS):
        cb = cb_ref[:]
        b2 = jnp.sum(cb * cb, axis=1)[None, :]
        for c in range(_N // _CHUNK):
            rows = data_ref[pl.ds(c * _CHUNK, _CHUNK), :]
            a2 = jnp.sum(rows * rows, axis=1, keepdims=True)
            prod = jax.lax.dot_general(
                rows, cb, (((1,), (1,)), ((), ())),
                preferred_element_type=jnp.float32)
            d2 = a2 + b2 - 2.0 * prod
            dist_ref[pl.ds(c * _CHUNK, _CHUNK), :] = jnp.sqrt(
                jnp.maximum(d2, 0.0))

        counts_f = jnp.zeros((1, _K), jnp.float32)
        for w in range(_N // _BW):
            base = w * _BW
            win0 = dist_ref[pl.ds(base, _BW), :]
            open_f = jnp.where(counts_f < capf, 0.0, jnp.float32(jnp.inf))
            idx0 = jnp.argmin(win0 + open_f, axis=1).astype(jnp.int32)[:, None]
            oh0 = jnp.where(lane == idx0, 1.0, 0.0).astype(jnp.bfloat16)
            oh_ref[:] = oh0
            prefix0 = jax.lax.dot_general(
                tri_ref[:], oh0, (((1,), (0,)), ((), ())),
                preferred_element_type=jnp.float32)
            pre_ref[:] = prefix0
            rem = capf - counts_f
            ok0 = jnp.where(lane == idx0,
                            jnp.where(prefix0 > rem, 0, 1), 1)
            done0 = jnp.min(ok0)

            def fix_cond(carry):
                return carry[1] == 0

            def fix_body(carry):
                idx, _ = carry
                win = dist_ref[pl.ds(base, _BW), :]
                pre_excl = pre_ref[:] - oh_ref[:].astype(jnp.float32)
                open_t = (counts_f + pre_excl) < capf
                idx2 = jnp.argmin(jnp.where(open_t, win, jnp.float32(jnp.inf)),
                                  axis=1).astype(jnp.int32)[:, None]
                oh2 = jnp.where(lane == idx2, 1.0, 0.0).astype(jnp.bfloat16)
                oh_ref[:] = oh2
                pre_ref[:] = jax.lax.dot_general(
                    tri_ref[:], oh2, (((1,), (0,)), ((), ())),
                    preferred_element_type=jnp.float32)
                done2 = jnp.min(jnp.where(idx2 == idx, 1, 0))
                return idx2, done2

            idx_fin, _ = jax.lax.while_loop(fix_cond, fix_body, (idx0, done0))
            labels_ref[pl.ds(base, _BW), :] = idx_fin
            counts_f = counts_f + pre_ref[pl.ds(_BW - 1, 1), :]

        sums_ref[:] = jnp.zeros((_K, _D), jnp.float32)
        for c in range(_N // _CHUNK):
            lbl = labels_ref[pl.ds(c * _CHUNK, _CHUNK), :]
            onehot = (lbl == jax.lax.broadcasted_iota(
                jnp.int32, (_CHUNK, _K), 1)).astype(jnp.bfloat16)
            rows = data_ref[pl.ds(c * _CHUNK, _CHUNK), :]
            hi = rows.astype(jnp.bfloat16)
            r1 = rows - hi.astype(jnp.float32)
            mid = r1.astype(jnp.bfloat16)
            lo = (r1 - mid.astype(jnp.float32)).astype(jnp.bfloat16)
            acc = sums_ref[:]
            for part in (hi, mid, lo):
                acc = acc + jax.lax.dot_general(
                    onehot, part, (((0,), (0,)), ((), ())),
                    preferred_element_type=jnp.float32)
            sums_ref[:] = acc
        cb_ref[:] = sums_ref[:] / capf


def kernel(data):
    n = data.shape[0]
    perm = jax.random.permutation(jax.random.key(1), n)[:_K]
    cb0 = data[perm]
    cb, labels = pl.pallas_call(
        _kmeans_kernel,
        out_shape=[
            jax.ShapeDtypeStruct((_K, _D), jnp.float32),
            jax.ShapeDtypeStruct((_N, 1), jnp.int32),
        ],
        scratch_shapes=[
            pltpu.VMEM((_N, _K), jnp.float32),
            pltpu.VMEM((_K, _D), jnp.float32),
            pltpu.VMEM((_BW, _K), jnp.bfloat16),
            pltpu.VMEM((_BW, _K), jnp.float32),
            pltpu.VMEM((_BW, _BW), jnp.bfloat16),
        ],
    )(data, cb0)
    return cb, labels.reshape(n)
